# Initial kernel scaffold; baseline (speedup 1.0000x reference)
import jax
import jax.numpy as jnp
from jax.experimental import pallas as pl


def kernel(x, edge_index, edge_weight, batch, W1, b1, g1, be1, W2, b2, g2, be2, W3, b3, g3, be3):
    raise NotImplementedError("write your pallas kernel here")


# speedup vs baseline: 8.8552x; 8.8552x over previous
import functools

import jax
import jax.numpy as jnp
from jax import lax
from jax.experimental import pallas as pl
from jax.experimental.pallas import tpu as pltpu
from jax.experimental.pallas import tpu_sc as plsc

NC = 2
NS = 16
NW = NC * NS

N = 10000
E = 320000
D = 128
DH = D // NC

CHUNK = 80
DEG_EPT = E // NW
DEG_NCHUNK = DEG_EPT // CHUNK
EPT = E // NS
NCHUNK = EPT // CHUNK
SPAN = 624
LAST_SPAN = N - SPAN * (NS - 1)
ZROWS = 208

_sc_mesh = plsc.VectorSubcoreMesh(core_axis_name="c", subcore_axis_name="s")


def _deg_body(dstr, ewr, degp, dst_v, ew_v, zb_v, deg_sh):
    c = lax.axis_index("c")
    s = lax.axis_index("s")
    wid = s * NC + c
    pltpu.sync_copy(dstr.at[wid], dst_v)
    pltpu.sync_copy(ewr.at[wid], ew_v)
    for k in range(LAST_SPAN // 16):
        zb_v[pl.ds(k * 16, 16)] = jnp.zeros((16,), jnp.float32)
    base = SPAN * s

    @pl.when(s < NS - 1)
    def _():
        pltpu.sync_copy(zb_v.at[pl.ds(0, SPAN)], deg_sh.at[pl.ds(base, SPAN)])

    @pl.when(s == NS - 1)
    def _():
        pltpu.sync_copy(zb_v, deg_sh.at[pl.ds(SPAN * (NS - 1), LAST_SPAN)])

    plsc.subcore_barrier()

    def chunk_body(i, carry):
        pltpu.sync_copy(ew_v.at[i], deg_sh.at[dst_v.at[i]], add=True)
        return carry

    lax.fori_loop(0, DEG_NCHUNK, chunk_body, 0)
    plsc.subcore_barrier()

    @pl.when(s < NS - 1)
    def _():
        pltpu.sync_copy(deg_sh.at[pl.ds(base, SPAN)], zb_v.at[pl.ds(0, SPAN)])
        pltpu.sync_copy(zb_v.at[pl.ds(0, SPAN)],
                        degp.at[pl.ds(c * N + base, SPAN)])

    @pl.when(s == NS - 1)
    def _():
        pltpu.sync_copy(deg_sh.at[pl.ds(SPAN * (NS - 1), LAST_SPAN)], zb_v)
        pltpu.sync_copy(zb_v,
                        degp.at[pl.ds(c * N + SPAN * (NS - 1), LAST_SPAN)])


_sc_deg = functools.partial(
    pl.kernel,
    out_type=jax.ShapeDtypeStruct((NC * N,), jnp.float32),
    mesh=_sc_mesh,
    scratch_types=[
        pltpu.VMEM((DEG_NCHUNK, CHUNK), jnp.int32),
        pltpu.VMEM((DEG_NCHUNK, CHUNK), jnp.float32),
        pltpu.VMEM((LAST_SPAN,), jnp.float32),
        pltpu.VMEM_SHARED((N,), jnp.float32),
    ],
)(_deg_body)


def _prop_body(hs2, srcr, dstr, ewr, part, src_v, dst_v, ew_v, rows_v, zb_v,
               acc_sh):
    c = lax.axis_index("c")
    s = lax.axis_index("s")
    pltpu.sync_copy(srcr.at[s], src_v)
    pltpu.sync_copy(dstr.at[s], dst_v)
    pltpu.sync_copy(ewr.at[s], ew_v)

    cn = c * N

    def adj(r, carry):
        for g in range(CHUNK // 16):
            sl = pl.ds(g * 16, 16)
            src_v[r, sl] = src_v[r, sl] + cn
        return carry

    lax.fori_loop(0, NCHUNK, adj, 0)

    def zrow(r, carry):
        for j in range(DH // 16):
            zb_v[r, pl.ds(j * 16, 16)] = jnp.zeros((16,), jnp.float32)
        return carry

    lax.fori_loop(0, ZROWS, zrow, 0)
    base = SPAN * s
    for k in range(SPAN // ZROWS):
        pltpu.sync_copy(zb_v, acc_sh.at[pl.ds(base + k * ZROWS, ZROWS)])

    @pl.when(s == NS - 1)
    def _():
        tail = LAST_SPAN - SPAN
        pltpu.sync_copy(zb_v.at[pl.ds(0, tail)],
                        acc_sh.at[pl.ds(N - tail, tail)])

    plsc.subcore_barrier()

    def chunk_body(i, carry):
        pltpu.sync_copy(hs2.at[src_v.at[i]], rows_v)

        def edge_group(g, carry2):
            wv = ew_v[i, pl.ds(g * 16, 16)]
            for l in range(16):
                w = wv[l]
                e = g * 16 + l
                for j in range(DH // 16):
                    sl = pl.ds(j * 16, 16)
                    rows_v[e, sl] = rows_v[e, sl] * w
            return carry2

        lax.fori_loop(0, CHUNK // 16, edge_group, 0)
        pltpu.sync_copy(rows_v, acc_sh.at[dst_v.at[i]], add=True)
        return carry

    lax.fori_loop(0, NCHUNK, chunk_body, 0)
    plsc.subcore_barrier()

    for k in range(SPAN // ZROWS):
        pltpu.sync_copy(acc_sh.at[pl.ds(base + k * ZROWS, ZROWS)], zb_v)
        pltpu.sync_copy(zb_v, part.at[pl.ds(cn + base + k * ZROWS, ZROWS)])

    @pl.when(s == NS - 1)
    def _():
        tail = LAST_SPAN - SPAN
        pltpu.sync_copy(acc_sh.at[pl.ds(N - tail, tail)],
                        zb_v.at[pl.ds(0, tail)])
        pltpu.sync_copy(zb_v.at[pl.ds(0, tail)],
                        part.at[pl.ds(cn + N - tail, tail)])


_sc_prop = functools.partial(
    pl.kernel,
    out_type=jax.ShapeDtypeStruct((NC * N, DH), jnp.float32),
    mesh=_sc_mesh,
    scratch_types=[
        pltpu.VMEM((NCHUNK, CHUNK), jnp.int32),
        pltpu.VMEM((NCHUNK, CHUNK), jnp.int32),
        pltpu.VMEM((NCHUNK, CHUNK), jnp.float32),
        pltpu.VMEM((CHUNK, DH), jnp.float32),
        pltpu.VMEM((ZROWS, DH), jnp.float32),
        pltpu.VMEM_SHARED((N, DH), jnp.float32),
    ],
    compiler_params=pltpu.CompilerParams(use_tc_tiling_on_sc=False),
)(_prop_body)


def _split2(y):
    return jnp.stack([y[:, :DH], y[:, DH:]], axis=0)


def _join2(p):
    return jnp.concatenate([p[0], p[1]], axis=1)


def _t0_body(x_ref, w_ref, degp_ref, dis_ref, hs2_ref):
    degp = degp_ref[...]
    deg = degp[0:1, :] + degp[1:2, :] + 1.0
    dis_row = lax.rsqrt(deg)
    dis_col = jnp.transpose(dis_row)
    dis_ref[...] = dis_col
    h = jnp.dot(x_ref[...], w_ref[...], preferred_element_type=jnp.float32)
    hs2_ref[...] = _split2(h * dis_col)


def _t0(x, w, degp):
    return pl.pallas_call(
        _t0_body,
        out_shape=(
            jax.ShapeDtypeStruct((N, 1), jnp.float32),
            jax.ShapeDtypeStruct((NC, N, DH), jnp.float32),
        ),
    )(x, w, degp)


BLK = 400
NBLK = N // BLK


def _conv_out(p_ref, hs2_ref, dis_ref, b_ref):
    y = _join2(p_ref[...] + hs2_ref[...])
    return y * dis_ref[...] + b_ref[...]


def _stats_body(p_ref, hs2_ref, dis_ref, b_ref, sums_ref):
    j = pl.program_id(0)
    y = _conv_out(p_ref, hs2_ref, dis_ref, b_ref)

    @pl.when(j == 0)
    def _():
        sums_ref[...] = jnp.zeros_like(sums_ref)

    s1 = jnp.sum(y, axis=0, keepdims=True)
    s2 = jnp.sum(y * y, axis=0, keepdims=True)
    sums_ref[...] += jnp.concatenate([s1, s2], axis=0)


def _t_stats(p, hs2, dis, b):
    return pl.pallas_call(
        _stats_body,
        grid=(NBLK,),
        in_specs=[
            pl.BlockSpec((NC, BLK, DH), lambda j: (0, j, 0)),
            pl.BlockSpec((NC, BLK, DH), lambda j: (0, j, 0)),
            pl.BlockSpec((BLK, 1), lambda j: (j, 0)),
            pl.BlockSpec((1, D), lambda j: (0, 0)),
        ],
        out_specs=pl.BlockSpec((2, D), lambda j: (0, 0)),
        out_shape=jax.ShapeDtypeStruct((2, D), jnp.float32),
    )(p, hs2, dis, b)


def _bn(y, sums_ref, g_ref, be_ref):
    mean = sums_ref[0:1, :] * (1.0 / N)
    var = sums_ref[1:2, :] * (1.0 / N) - mean * mean
    return (y - mean) * lax.rsqrt(var + 1e-5) * g_ref[...] + be_ref[...]


def _fin_body(p_ref, hs2_ref, dis_ref, b_ref, g_ref, be_ref, sums_ref, w_ref,
              out_ref):
    y = _conv_out(p_ref, hs2_ref, dis_ref, b_ref)
    z = jnp.maximum(_bn(y, sums_ref, g_ref, be_ref), 0.0)
    h = jnp.dot(z, w_ref[...], preferred_element_type=jnp.float32)
    out_ref[...] = _split2(h * dis_ref[...])


def _t_fin(p, hs2, dis, b, g, be, sums, w_next):
    return pl.pallas_call(
        _fin_body,
        grid=(NBLK,),
        in_specs=[
            pl.BlockSpec((NC, BLK, DH), lambda j: (0, j, 0)),
            pl.BlockSpec((NC, BLK, DH), lambda j: (0, j, 0)),
            pl.BlockSpec((BLK, 1), lambda j: (j, 0)),
            pl.BlockSpec((1, D), lambda j: (0, 0)),
            pl.BlockSpec((1, D), lambda j: (0, 0)),
            pl.BlockSpec((1, D), lambda j: (0, 0)),
            pl.BlockSpec((2, D), lambda j: (0, 0)),
            pl.BlockSpec((D, D), lambda j: (0, 0)),
        ],
        out_specs=pl.BlockSpec((NC, BLK, DH), lambda j: (0, j, 0)),
        out_shape=jax.ShapeDtypeStruct((NC, N, DH), jnp.float32),
    )(p, hs2, dis, b, g, be, sums, w_next)


def _fin_last_body(p_ref, hs2_ref, dis_ref, b_ref, g_ref, be_ref, sums_ref,
                   out_ref):
    y = _conv_out(p_ref, hs2_ref, dis_ref, b_ref)
    out_ref[...] = _bn(y, sums_ref, g_ref, be_ref)


def _t_fin_last(p, hs2, dis, b, g, be, sums):
    return pl.pallas_call(
        _fin_last_body,
        grid=(NBLK,),
        in_specs=[
            pl.BlockSpec((NC, BLK, DH), lambda j: (0, j, 0)),
            pl.BlockSpec((NC, BLK, DH), lambda j: (0, j, 0)),
            pl.BlockSpec((BLK, 1), lambda j: (j, 0)),
            pl.BlockSpec((1, D), lambda j: (0, 0)),
            pl.BlockSpec((1, D), lambda j: (0, 0)),
            pl.BlockSpec((1, D), lambda j: (0, 0)),
            pl.BlockSpec((2, D), lambda j: (0, 0)),
        ],
        out_specs=pl.BlockSpec((BLK, D), lambda j: (j, 0)),
        out_shape=jax.ShapeDtypeStruct((N, D), jnp.float32),
    )(p, hs2, dis, b, g, be, sums)


def kernel(x, edge_index, edge_weight, batch, W1, b1, g1, be1, W2, b2, g2,
           be2, W3, b3, g3, be3):
    src = edge_index[0]
    dst = edge_index[1]
    dstr32 = jnp.reshape(dst, (NW, DEG_NCHUNK, CHUNK))
    ewr32 = jnp.reshape(edge_weight, (NW, DEG_NCHUNK, CHUNK))
    srcr = jnp.reshape(src, (NS, NCHUNK, CHUNK))
    dstr = jnp.reshape(dst, (NS, NCHUNK, CHUNK))
    ewr = jnp.reshape(edge_weight, (NS, NCHUNK, CHUNK))

    degp = jnp.reshape(_sc_deg(dstr32, ewr32), (NC, N))
    dis, hs2 = _t0(x, W1, degp)

    layers = [(b1, g1, be1, W2), (b2, g2, be2, W3), (b3, g3, be3, None)]
    out = None
    for b, g, be, w_next in layers:
        br = jnp.reshape(b, (1, D))
        gr = jnp.reshape(g, (1, D))
        ber = jnp.reshape(be, (1, D))
        hs_flat = jnp.reshape(hs2, (NC * N, DH))
        p = jnp.reshape(_sc_prop(hs_flat, srcr, dstr, ewr), (NC, N, DH))
        sums = _t_stats(p, hs2, dis, br)
        if w_next is not None:
            hs2 = _t_fin(p, hs2, dis, br, gr, ber, sums, w_next)
        else:
            out = _t_fin_last(p, hs2, dis, br, gr, ber, sums)
    return out
